# baseline (speedup 1.0000x reference)
import functools

import jax
import jax.numpy as jnp
from jax import lax
from jax.experimental import pallas as pl
from jax.experimental.pallas import tpu as pltpu
from jax.experimental.pallas import tpu_sc as plsc

_NUM_CORES = 2
_NUM_SUBCORES = 16
_NW = _NUM_CORES * _NUM_SUBCORES
_L = 16
_H = 128


def _embed_t(xt, tab2):
    s_len, b_len = xt.shape
    v2, dd = tab2.shape
    d = dd // 2
    w_b = b_len // _NW
    n_steps = s_len * (w_b // _H)

    mesh = plsc.VectorSubcoreMesh(
        core_axis_name="c", subcore_axis_name="s",
        num_cores=_NUM_CORES, num_subcores=_NUM_SUBCORES)

    @functools.partial(
        pl.kernel,
        mesh=mesh,
        compiler_params=pltpu.CompilerParams(
            use_tc_tiling_on_sc=True, needs_layout_passes=False),
        out_type=jax.ShapeDtypeStruct((s_len, d, b_len), jnp.float32),
        scratch_types=[
            pltpu.VMEM((s_len, w_b), jnp.int32),
            pltpu.VMEM((2, _H), jnp.int32),
            pltpu.VMEM((2, _H), jnp.int32),
            pltpu.VMEM((2, _H, dd), jnp.float32),
            pltpu.VMEM((2, 1, d, _H), jnp.float32),
            pltpu.SemaphoreType.DMA,
            pltpu.SemaphoreType.DMA,
            pltpu.SemaphoreType.DMA,
            pltpu.SemaphoreType.DMA,
        ],
    )
    def emb(tab_hbm, xt_hbm, out_hbm, x_v, idx_u, par, buf, buf_t,
            gsem0, gsem1, osem0, osem1):
        wid = lax.axis_index("s") * _NUM_CORES + lax.axis_index("c")
        b0 = wid * w_b
        gsems = (gsem0, gsem1)
        osems = (osem0, osem1)
        iot = lax.iota(jnp.int32, _L)

        pltpu.sync_copy(xt_hbm.at[:, pl.ds(b0, w_b)], x_v)

        spb = w_b // _H

        def prep(step, slot):
            s = step // spb
            h = step % spb

            def jbody(j, _):
                v = x_v[s, pl.ds(h * _H + j * _L, _L)]
                idx_u[slot, pl.ds(j * _L, _L)] = v >> 1
                par[slot, pl.ds(j * _L, _L)] = v & 1
                return 0

            lax.fori_loop(0, _H // _L, jbody, 0)
            pltpu.async_copy(
                tab_hbm.at[idx_u.at[slot]], buf.at[slot], gsems[slot])

        prep(0, 0)

        def body2(q, _):
            for slot in range(2):
                k = q * 2 + slot

                @pl.when(k + 1 < n_steps)
                def _start_next():
                    prep(k + 1, 1 - slot)

                pltpu.make_async_copy(
                    tab_hbm.at[idx_u.at[slot]], buf.at[slot],
                    gsems[slot]).wait()

                s = k // spb
                h = k % spb
                bb = b0 + h * _H

                @pl.when(k >= 2)
                def _drain_out():
                    pltpu.make_async_copy(
                        buf_t.at[slot],
                        out_hbm.at[pl.ds(s, 1), :, pl.ds(bb, _H)],
                        osems[slot]).wait()

                def tbody(j, _):
                    row = j * _L + iot
                    colb = par[slot, pl.ds(j * _L, _L)] * d
                    for dg in range(d):
                        g = plsc.load_gather(
                            buf.at[slot], [row, colb + dg])
                        buf_t[slot, 0, dg, pl.ds(j * _L, _L)] = g
                    return 0

                lax.fori_loop(0, _H // _L, tbody, 0)

                pltpu.async_copy(
                    buf_t.at[slot],
                    out_hbm.at[pl.ds(s, 1), :, pl.ds(bb, _H)],
                    osems[slot])
            return 0

        lax.fori_loop(0, n_steps // 2, body2, 0)

        for slot in range(2):
            k = n_steps - 2 + slot
            s = k // spb
            h = k % spb
            pltpu.make_async_copy(
                buf_t.at[slot],
                out_hbm.at[pl.ds(s, 1), :, pl.ds(b0 + h * _H, _H)],
                osems[slot]).wait()

    return emb(tab2, xt)


def kernel(x, table):
    b, s = x.shape
    v, d = table.shape
    xt = x.T.astype(jnp.int32)
    tab2 = table.reshape(v // 2, 2 * d)
    out_t = _embed_t(xt, tab2)
    return out_t.transpose(2, 0, 1)


# speedup vs baseline: 1.3017x; 1.3017x over previous
import functools

import jax
import jax.numpy as jnp
from jax import lax
from jax.experimental import pallas as pl
from jax.experimental.pallas import tpu as pltpu
from jax.experimental.pallas import tpu_sc as plsc

_NUM_CORES = 2
_NUM_SUBCORES = 16
_NW = _NUM_CORES * _NUM_SUBCORES
_L = 16
_H = 128


def _embed_t(xt, tab2):
    s_len, b_len = xt.shape
    v2, dd = tab2.shape
    d = dd // 2
    w_b = b_len // _NW
    n_steps = s_len * (w_b // _H)

    mesh = plsc.VectorSubcoreMesh(
        core_axis_name="c", subcore_axis_name="s",
        num_cores=_NUM_CORES, num_subcores=_NUM_SUBCORES)

    @functools.partial(
        pl.kernel,
        mesh=mesh,
        compiler_params=pltpu.CompilerParams(
            use_tc_tiling_on_sc=True, needs_layout_passes=False),
        out_type=jax.ShapeDtypeStruct((s_len, d, b_len), jnp.float32),
        scratch_types=[
            pltpu.VMEM((s_len, w_b), jnp.int32),
            pltpu.VMEM((2, _H), jnp.int32),
            pltpu.VMEM((2, _H), jnp.int32),
            pltpu.VMEM((2, _H, dd), jnp.float32),
            pltpu.VMEM((2, 1, d, _H), jnp.float32),
            pltpu.SemaphoreType.DMA,
            pltpu.SemaphoreType.DMA,
            pltpu.SemaphoreType.DMA,
            pltpu.SemaphoreType.DMA,
        ],
    )
    def emb(tab_hbm, xt_hbm, out_hbm, x_v, idx_u, par, buf, buf_t,
            gsem0, gsem1, osem0, osem1):
        wid = lax.axis_index("s") * _NUM_CORES + lax.axis_index("c")
        b0 = wid * w_b
        gsems = (gsem0, gsem1)
        osems = (osem0, osem1)
        iot = lax.iota(jnp.int32, _L)

        pltpu.sync_copy(xt_hbm.at[:, pl.ds(b0, w_b)], x_v)

        spb = w_b // _H

        def prep(step, slot):
            s = step // spb
            h = step % spb

            def jbody(j, _):
                v = x_v[s, pl.ds(h * _H + j * _L, _L)]
                idx_u[slot, pl.ds(j * _L, _L)] = v >> 1
                par[slot, pl.ds(j * _L, _L)] = v & 1
                return 0

            lax.fori_loop(0, _H // _L, jbody, 0)
            pltpu.async_copy(
                tab_hbm.at[idx_u.at[slot]], buf.at[slot], gsems[slot])

        prep(0, 0)

        def body2(q, _):
            for slot in range(2):
                k = q * 2 + slot

                @pl.when(k + 1 < n_steps)
                def _start_next():
                    prep(k + 1, 1 - slot)

                pltpu.make_async_copy(
                    tab_hbm.at[idx_u.at[slot]], buf.at[slot],
                    gsems[slot]).wait()

                s = k // spb
                h = k % spb
                bb = b0 + h * _H

                @pl.when(k >= 2)
                def _drain_out():
                    pltpu.make_async_copy(
                        buf_t.at[slot],
                        out_hbm.at[pl.ds(s, 1), :, pl.ds(bb, _H)],
                        osems[slot]).wait()

                @plsc.parallel_loop(0, _H // _L, unroll=2)
                def tbody(j):
                    row = j * _L + iot
                    colb = par[slot, pl.ds(j * _L, _L)] * d
                    for dg in range(d):
                        g = plsc.load_gather(
                            buf.at[slot], [row, colb + dg])
                        buf_t[slot, 0, dg, pl.ds(j * _L, _L)] = g

                pltpu.async_copy(
                    buf_t.at[slot],
                    out_hbm.at[pl.ds(s, 1), :, pl.ds(bb, _H)],
                    osems[slot])
            return 0

        lax.fori_loop(0, n_steps // 2, body2, 0)

        for slot in range(2):
            k = n_steps - 2 + slot
            s = k // spb
            h = k % spb
            pltpu.make_async_copy(
                buf_t.at[slot],
                out_hbm.at[pl.ds(s, 1), :, pl.ds(b0 + h * _H, _H)],
                osems[slot]).wait()

    return emb(tab2, xt)


def kernel(x, table):
    b, s = x.shape
    v, d = table.shape
    xt = x.T.astype(jnp.int32)
    tab2 = table.reshape(v // 2, 2 * d)
    out_t = _embed_t(xt, tab2)
    return out_t.transpose(2, 0, 1)


# speedup vs baseline: 1.3581x; 1.0434x over previous
import functools

import jax
import jax.numpy as jnp
from jax import lax
from jax.experimental import pallas as pl
from jax.experimental.pallas import tpu as pltpu
from jax.experimental.pallas import tpu_sc as plsc

_NUM_CORES = 2
_NUM_SUBCORES = 16
_NW = _NUM_CORES * _NUM_SUBCORES
_L = 16
_H = 256


def _embed_t(xt, table):
    s_len, b_len = xt.shape
    v_len, d = table.shape
    w_b = b_len // _NW
    spb = w_b // _H
    n_steps = s_len * spb
    dg8 = d // 8
    bt = _H // 128

    mesh = plsc.VectorSubcoreMesh(
        core_axis_name="c", subcore_axis_name="s",
        num_cores=_NUM_CORES, num_subcores=_NUM_SUBCORES)

    @functools.partial(
        pl.kernel,
        mesh=mesh,
        compiler_params=pltpu.CompilerParams(
            use_tc_tiling_on_sc=False, needs_layout_passes=False),
        out_type=jax.ShapeDtypeStruct(
            (s_len, dg8, b_len // 128, 8, 128), jnp.float32),
        scratch_types=[
            pltpu.VMEM((s_len, w_b), jnp.int32),
            pltpu.VMEM((2, _H, d), jnp.float32),
            pltpu.VMEM((2, 1, dg8, bt, 8, 128), jnp.float32),
            pltpu.SemaphoreType.DMA,
            pltpu.SemaphoreType.DMA,
            pltpu.SemaphoreType.DMA,
            pltpu.SemaphoreType.DMA,
        ],
    )
    def emb(tab_hbm, xt_hbm, out_hbm, x_v, buf, buf_t,
            gsem0, gsem1, osem0, osem1):
        wid = lax.axis_index("s") * _NUM_CORES + lax.axis_index("c")
        b0 = wid * w_b
        bt0 = b0 // 128
        gsems = (gsem0, gsem1)
        osems = (osem0, osem1)
        iot = lax.iota(jnp.int32, _L)

        pltpu.sync_copy(xt_hbm.at[:, pl.ds(b0, w_b)], x_v)

        def start_gather(step, slot):
            s = step // spb
            h = step % spb
            pltpu.async_copy(
                tab_hbm.at[x_v.at[s, pl.ds(h * _H, _H)]],
                buf.at[slot], gsems[slot])

        def wait_gather(slot):
            pltpu.make_async_copy(
                tab_hbm.at[x_v.at[0, pl.ds(0, _H)]],
                buf.at[slot], gsems[slot]).wait()

        def out_dma(step, slot):
            s = step // spb
            h = step % spb
            return pltpu.make_async_copy(
                buf_t.at[slot],
                out_hbm.at[pl.ds(s, 1), :, pl.ds(bt0 + h * bt, bt)],
                osems[slot])

        start_gather(0, 0)

        def body2(q, _):
            for slot in range(2):
                k = q * 2 + slot

                @pl.when(k + 1 < n_steps)
                def _start_next():
                    start_gather(k + 1, 1 - slot)

                wait_gather(slot)

                @pl.when(k >= 2)
                def _drain_out():
                    out_dma(k, slot).wait()

                @plsc.parallel_loop(0, _H // _L, unroll=2)
                def tbody(j):
                    row = j * _L + iot
                    t = j // (128 // _L)
                    l0 = (j % (128 // _L)) * _L
                    for g in range(dg8):
                        for r in range(8):
                            v = plsc.load_gather(
                                buf.at[slot], [row, iot * 0 + (g * 8 + r)])
                            buf_t[slot, 0, g, t, r, pl.ds(l0, _L)] = v

                out_dma(k, slot).start()
            return 0

        lax.fori_loop(0, n_steps // 2, body2, 0)

        for slot in range(2):
            out_dma(n_steps - 2 + slot, slot).wait()

    return emb(table, xt)


def kernel(x, table):
    b, s = x.shape
    v, d = table.shape
    xt = x.T.astype(jnp.int32)
    out5 = _embed_t(xt, table)
    out = out5.transpose(2, 4, 0, 1, 3).reshape(b, s, d)
    return out


# speedup vs baseline: 1.4226x; 1.0475x over previous
import functools

import jax
import jax.numpy as jnp
from jax import lax
from jax.experimental import pallas as pl
from jax.experimental.pallas import tpu as pltpu
from jax.experimental.pallas import tpu_sc as plsc

_NUM_CORES = 2
_NUM_SUBCORES = 16
_NW = _NUM_CORES * _NUM_SUBCORES
_L = 16
_H = 256


def _embed_t(xt, table):
    s_len, b_len = xt.shape
    v_len, dd = table.shape
    d = dd // 2
    w_b = b_len // _NW
    spb = w_b // _H
    n_steps = s_len * spb
    dg8 = d // 8
    bt = _H // 128

    mesh = plsc.VectorSubcoreMesh(
        core_axis_name="c", subcore_axis_name="s",
        num_cores=_NUM_CORES, num_subcores=_NUM_SUBCORES)

    @functools.partial(
        pl.kernel,
        mesh=mesh,
        compiler_params=pltpu.CompilerParams(
            use_tc_tiling_on_sc=False, needs_layout_passes=False),
        out_type=jax.ShapeDtypeStruct(
            (s_len, dg8, b_len // 128, 8, 128), jnp.float32),
        scratch_types=[
            pltpu.VMEM((s_len, w_b), jnp.int32),
            pltpu.VMEM((2, _H, dd), jnp.float32),
            pltpu.VMEM((2, 1, dg8, bt, 8, 128), jnp.float32),
            pltpu.SemaphoreType.DMA,
            pltpu.SemaphoreType.DMA,
            pltpu.SemaphoreType.DMA,
            pltpu.SemaphoreType.DMA,
        ],
    )
    def emb(tab_hbm, xt_hbm, out_hbm, x_v, buf, buf_t,
            gsem0, gsem1, osem0, osem1):
        wid = lax.axis_index("s") * _NUM_CORES + lax.axis_index("c")
        b0 = wid * w_b
        bt0 = b0 // 128
        gsems = (gsem0, gsem1)
        osems = (osem0, osem1)
        iot = lax.iota(jnp.int32, _L)

        pltpu.sync_copy(xt_hbm.at[:, pl.ds(b0, w_b)], x_v)

        def start_gather(step, slot):
            s = step // spb
            h = step % spb
            pltpu.async_copy(
                tab_hbm.at[x_v.at[s, pl.ds(h * _H, _H)]],
                buf.at[slot], gsems[slot])

        def wait_gather(slot):
            pltpu.make_async_copy(
                tab_hbm.at[x_v.at[0, pl.ds(0, _H)]],
                buf.at[slot], gsems[slot]).wait()

        def out_dma(step, slot):
            s = step // spb
            h = step % spb
            return pltpu.make_async_copy(
                buf_t.at[slot],
                out_hbm.at[pl.ds(s, 1), :, pl.ds(bt0 + h * bt, bt)],
                osems[slot])

        start_gather(0, 0)

        def body2(q, _):
            for slot in range(2):
                k = q * 2 + slot

                @pl.when(k + 1 < n_steps)
                def _start_next():
                    start_gather(k + 1, 1 - slot)

                wait_gather(slot)

                @pl.when(k >= 2)
                def _drain_out():
                    out_dma(k, slot).wait()

                @plsc.parallel_loop(0, _H // _L, unroll=2)
                def tbody(j):
                    row = j * _L + iot
                    t = j // (128 // _L)
                    l0 = (j % (128 // _L)) * _L
                    for g in range(dg8):
                        for r in range(8):
                            v = plsc.load_gather(
                                buf.at[slot], [row, iot * 0 + (g * 8 + r)])
                            buf_t[slot, 0, g, t, r, pl.ds(l0, _L)] = v

                out_dma(k, slot).start()
            return 0

        lax.fori_loop(0, n_steps // 2, body2, 0)

        for slot in range(2):
            out_dma(n_steps - 2 + slot, slot).wait()

    return emb(table, xt)


def kernel(x, table):
    b, s = x.shape
    v, d = table.shape
    xt = x.T.astype(jnp.int32)
    tab_pad = jnp.pad(table, ((0, 0), (0, d)))
    out5 = _embed_t(xt, tab_pad)
    out = out5.transpose(2, 4, 0, 1, 3).reshape(b, s, d)
    return out
